# baseline (speedup 1.0000x reference)
import jax
import jax.numpy as jnp
from jax import lax
from jax.experimental import pallas as pl
from jax.experimental.pallas import tpu as pltpu

_NUM_SELECT = 300
_NP = 384
_C = 91
_ROWS = 80
_LANES = 1024
_QPAD = 1024
_IOU_THR = 0.5


def _topk_body(s_ref, vals_ref, q_ref, lab_ref, data):
    x = s_ref[0]
    data[...] = 1.0 / (1.0 + jnp.exp(-x))
    m0 = jnp.max(data[...], axis=1, keepdims=True)
    iota_r = lax.broadcasted_iota(jnp.int32, (_ROWS, 1), 0)
    iota_l = lax.broadcasted_iota(jnp.int32, (1, _LANES), 1)
    pos = (lax.broadcasted_iota(jnp.int32, (3, 128), 0) * 128
           + lax.broadcasted_iota(jnp.int32, (3, 128), 1))

    def body(i, carry):
        vacc, iacc, m = carry
        vmax = jnp.max(m)
        r = jnp.min(jnp.where(m == vmax, iota_r, _ROWS))
        row = data[pl.ds(r, 1), :]
        l = jnp.min(jnp.where(row == vmax, iota_l, _LANES))
        flat = r * _LANES + l
        selm = pos == i
        vacc = jnp.where(selm, vmax, vacc)
        iacc = jnp.where(selm, flat, iacc)
        row2 = jnp.where(iota_l == l, -1.0, row)
        data[pl.ds(r, 1), :] = row2
        m = jnp.where(iota_r == r, jnp.max(row2), m)
        return vacc, iacc, m

    vacc, iacc, _ = lax.fori_loop(
        0, _NUM_SELECT, body,
        (jnp.zeros((3, 128), jnp.float32), jnp.zeros((3, 128), jnp.int32), m0))
    vals_ref[0] = vacc
    q_ref[0] = iacc // _C
    lab_ref[0] = iacc % _C


def _nms_body(b_ref, bt_ref, qc_ref, qr_ref, sc_ref, boxes_ref, keep_ref, S):
    qc = qc_ref[0]
    qr = qr_ref[0]
    bb = b_ref[0]
    bt = bt_ref[0]
    sw = sc_ref[0, 0, 0]
    sh = sc_ref[0, 0, 1]

    onehot_c = (lax.broadcasted_iota(jnp.int32, (_NP, _QPAD), 1) == qc
                ).astype(jnp.float32)
    sel = jnp.dot(onehot_c, bb, preferred_element_type=jnp.float32,
                  precision=lax.Precision.HIGHEST)
    onehot_r = (lax.broadcasted_iota(jnp.int32, (_QPAD, _NP), 0) == qr
                ).astype(jnp.float32)
    selT = jnp.dot(bt, onehot_r, preferred_element_type=jnp.float32,
                   precision=lax.Precision.HIGHEST)

    cx, cy, w, h = sel[:, 0:1], sel[:, 1:2], sel[:, 2:3], sel[:, 3:4]
    x0 = (cx - 0.5 * w) * sw
    y0 = (cy - 0.5 * h) * sh
    x1 = (cx + 0.5 * w) * sw
    y1 = (cy + 0.5 * h) * sh
    cxT, cyT, wT, hT = selT[0:1], selT[1:2], selT[2:3], selT[3:4]
    x0T = (cxT - 0.5 * wT) * sw
    y0T = (cyT - 0.5 * hT) * sh
    x1T = (cxT + 0.5 * wT) * sw
    y1T = (cyT + 0.5 * hT) * sh

    area_c = (x1 - x0) * (y1 - y0)
    area_r = (x1T - x0T) * (y1T - y0T)
    ix0 = jnp.maximum(x0, x0T)
    iy0 = jnp.maximum(y0, y0T)
    ix1 = jnp.minimum(x1, x1T)
    iy1 = jnp.minimum(y1, y1T)
    inter = jnp.clip(ix1 - ix0, 0.0, None) * jnp.clip(iy1 - iy0, 0.0, None)
    union = area_c + area_r - inter
    iou = inter / jnp.clip(union, 1e-9, None)
    S[...] = (iou > _IOU_THR).astype(jnp.float32)
    boxes_ref[0] = jnp.concatenate([x0, y0, x1, y1], axis=1)

    jiota = lax.broadcasted_iota(jnp.int32, (1, _NP), 1)

    def body(i, keep):
        srow = S[pl.ds(i, 1), :]
        supp = jnp.max(jnp.where(jiota < i, keep * srow, 0.0))
        newk = jnp.where(supp > 0.0, 0.0, 1.0)
        return jnp.where(jiota == i, newk, keep)

    keep = lax.fori_loop(1, _NUM_SELECT, body, jnp.ones((1, _NP), jnp.float32))
    keep_ref[0] = keep.astype(jnp.int32)


def kernel(pred_logits, pred_boxes, target_sizes):
    B, Q, C = pred_logits.shape

    flat = pred_logits.reshape(B, Q * C)
    pad = _ROWS * _LANES - Q * C
    flatp = jnp.pad(flat, ((0, 0), (0, pad)), constant_values=-1e30)
    flatp = flatp.reshape(B, _ROWS, _LANES)

    vals, qidx, labs = pl.pallas_call(
        _topk_body,
        grid=(B,),
        in_specs=[pl.BlockSpec((1, _ROWS, _LANES), lambda b: (b, 0, 0))],
        out_specs=[pl.BlockSpec((1, 3, 128), lambda b: (b, 0, 0))] * 3,
        out_shape=[
            jax.ShapeDtypeStruct((B, 3, 128), jnp.float32),
            jax.ShapeDtypeStruct((B, 3, 128), jnp.int32),
            jax.ShapeDtypeStruct((B, 3, 128), jnp.int32),
        ],
        scratch_shapes=[pltpu.VMEM((_ROWS, _LANES), jnp.float32)],
        compiler_params=pltpu.CompilerParams(
            dimension_semantics=("parallel",)),
    )(flatp)

    q2 = qidx.reshape(B, _NP)
    qcol = q2.reshape(B, _NP, 1)
    qrow = q2.reshape(B, 1, _NP)
    bpad = jnp.pad(pred_boxes, ((0, 0), (0, _QPAD - Q), (0, 0)))
    bT = jnp.transpose(bpad, (0, 2, 1))
    img_h = target_sizes[:, 0].astype(jnp.float32)
    img_w = target_sizes[:, 1].astype(jnp.float32)
    scale = jnp.stack([img_w, img_h], axis=1).reshape(B, 1, 2)

    boxes_pad, keep_pad = pl.pallas_call(
        _nms_body,
        grid=(B,),
        in_specs=[
            pl.BlockSpec((1, _QPAD, 4), lambda b: (b, 0, 0)),
            pl.BlockSpec((1, 4, _QPAD), lambda b: (b, 0, 0)),
            pl.BlockSpec((1, _NP, 1), lambda b: (b, 0, 0)),
            pl.BlockSpec((1, 1, _NP), lambda b: (b, 0, 0)),
            pl.BlockSpec((1, 1, 2), lambda b: (b, 0, 0)),
        ],
        out_specs=[
            pl.BlockSpec((1, _NP, 4), lambda b: (b, 0, 0)),
            pl.BlockSpec((1, 1, _NP), lambda b: (b, 0, 0)),
        ],
        out_shape=[
            jax.ShapeDtypeStruct((B, _NP, 4), jnp.float32),
            jax.ShapeDtypeStruct((B, 1, _NP), jnp.int32),
        ],
        scratch_shapes=[pltpu.VMEM((_NP, _NP), jnp.float32)],
        compiler_params=pltpu.CompilerParams(
            dimension_semantics=("parallel",)),
    )(bpad, bT, qcol, qrow, scale)

    scores = vals.reshape(B, _NP)[:, :_NUM_SELECT]
    labels = labs.reshape(B, _NP)[:, :_NUM_SELECT]
    boxes = boxes_pad[:, :_NUM_SELECT, :]
    keep = keep_pad.reshape(B, _NP)[:, :_NUM_SELECT] != 0
    return scores, labels, boxes, keep


# speedup vs baseline: 2.3199x; 2.3199x over previous
import jax
import jax.numpy as jnp
from jax import lax
from jax.experimental import pallas as pl
from jax.experimental.pallas import tpu as pltpu

_NUM_SELECT = 300
_NP = 384
_C = 91
_RW = 320
_RL = 256
_QPAD = 1024
_IOU_THR = 0.5
_IMGS_PER_STEP = 8


def _sigmoid(x):
    return 1.0 / (1.0 + jnp.exp(-x))


def _topk_body(l_ref, lt_ref, vals_ref, idx_ref, data):
    data[...] = _sigmoid(l_ref[...])
    iota_r = lax.broadcasted_iota(jnp.int32, (1, _RW), 1)
    iota_l = lax.broadcasted_iota(jnp.int32, (1, _RL), 1)

    ms = [_sigmoid(jnp.max(lt_ref[img], axis=0, keepdims=True))
          for img in range(_IMGS_PER_STEP)]

    def body(i, carry):
        ms = list(carry)
        for img in range(_IMGS_PER_STEP):
            m = ms[img]
            vmax = jnp.max(m, axis=1, keepdims=True)
            r = jnp.min(jnp.where(m == vmax, iota_r, _RW))
            row = data[img, pl.ds(r, 1), :]
            lvec = jnp.min(jnp.where(row == vmax, iota_l, _RL),
                           axis=1, keepdims=True)
            vals_ref[img, pl.ds(i, 1), :] = vmax
            idx_ref[img, pl.ds(i, 1), :] = r * _RL + lvec
            row2 = jnp.where(iota_l == lvec, -1.0, row)
            data[img, pl.ds(r, 1), :] = row2
            ms[img] = jnp.where(iota_r == r,
                                jnp.max(row2, axis=1, keepdims=True), m)
        return tuple(ms)

    lax.fori_loop(0, _NUM_SELECT, body, tuple(ms))


def _iou_body(b_ref, bt_ref, ic_ref, ir_ref, sc_ref, boxes_ref, s_ref,
              lab_ref):
    qc = ic_ref[0] // _C
    qr = ir_ref[0] // _C
    lab_ref[0] = ir_ref[0] % _C
    bb = b_ref[0]
    bt = bt_ref[0]
    sw = sc_ref[0, 0, 0]
    sh = sc_ref[0, 0, 1]

    onehot_c = (lax.broadcasted_iota(jnp.int32, (_NP, _QPAD), 1) == qc
                ).astype(jnp.float32)
    sel = jnp.dot(onehot_c, bb, preferred_element_type=jnp.float32,
                  precision=lax.Precision.HIGHEST)
    onehot_r = (lax.broadcasted_iota(jnp.int32, (_QPAD, _NP), 0) == qr
                ).astype(jnp.float32)
    selT = jnp.dot(bt, onehot_r, preferred_element_type=jnp.float32,
                   precision=lax.Precision.HIGHEST)

    cx, cy, w, h = sel[:, 0:1], sel[:, 1:2], sel[:, 2:3], sel[:, 3:4]
    x0 = (cx - 0.5 * w) * sw
    y0 = (cy - 0.5 * h) * sh
    x1 = (cx + 0.5 * w) * sw
    y1 = (cy + 0.5 * h) * sh
    cxT, cyT, wT, hT = selT[0:1], selT[1:2], selT[2:3], selT[3:4]
    x0T = (cxT - 0.5 * wT) * sw
    y0T = (cyT - 0.5 * hT) * sh
    x1T = (cxT + 0.5 * wT) * sw
    y1T = (cyT + 0.5 * hT) * sh

    area_c = (x1 - x0) * (y1 - y0)
    area_r = (x1T - x0T) * (y1T - y0T)
    ix0 = jnp.maximum(x0, x0T)
    iy0 = jnp.maximum(y0, y0T)
    ix1 = jnp.minimum(x1, x1T)
    iy1 = jnp.minimum(y1, y1T)
    inter = jnp.clip(ix1 - ix0, 0.0, None) * jnp.clip(iy1 - iy0, 0.0, None)
    union = area_c + area_r - inter
    iou = inter / jnp.clip(union, 1e-9, None)
    s_ref[0] = (iou > _IOU_THR).astype(jnp.float32)
    boxes_ref[0] = jnp.concatenate([x0, y0, x1, y1], axis=1)


def _nms_scan_body(s_ref, keep_ref):
    jiota = lax.broadcasted_iota(jnp.int32, (1, _NP), 1)
    ones = jnp.ones((1, _NP), jnp.float32)

    def body(i, keeps):
        keeps = list(keeps)
        for img in range(_IMGS_PER_STEP):
            keep = keeps[img]
            srow = s_ref[img, pl.ds(i, 1), :]
            supp = jnp.max(jnp.where(jiota < i, keep * srow, 0.0),
                           axis=1, keepdims=True)
            keeps[img] = jnp.where(jiota == i, 1.0 - supp, keep)
        return tuple(keeps)

    keeps = lax.fori_loop(1, _NUM_SELECT, body,
                          tuple(ones for _ in range(_IMGS_PER_STEP)))
    for img in range(_IMGS_PER_STEP):
        keep_ref[img] = keeps[img].astype(jnp.int32)


def kernel(pred_logits, pred_boxes, target_sizes):
    B, Q, C = pred_logits.shape
    nsteps = B // _IMGS_PER_STEP

    flat = pred_logits.reshape(B, Q * C)
    pad = _RW * _RL - Q * C
    flatp = jnp.pad(flat, ((0, 0), (0, pad)), constant_values=-1e30)
    lg = flatp.reshape(B, _RW, _RL)
    lgT = jnp.transpose(lg, (0, 2, 1))

    vals, idx = pl.pallas_call(
        _topk_body,
        grid=(nsteps,),
        in_specs=[
            pl.BlockSpec((_IMGS_PER_STEP, _RW, _RL), lambda g: (g, 0, 0)),
            pl.BlockSpec((_IMGS_PER_STEP, _RL, _RW), lambda g: (g, 0, 0)),
        ],
        out_specs=[pl.BlockSpec((_IMGS_PER_STEP, _NP, 1),
                                lambda g: (g, 0, 0))] * 2,
        out_shape=[
            jax.ShapeDtypeStruct((B, _NP, 1), jnp.float32),
            jax.ShapeDtypeStruct((B, _NP, 1), jnp.int32),
        ],
        scratch_shapes=[pltpu.VMEM((_IMGS_PER_STEP, _RW, _RL), jnp.float32)],
        compiler_params=pltpu.CompilerParams(
            dimension_semantics=("parallel",)),
    )(lg, lgT)

    i2 = idx.reshape(B, _NP)
    icol = i2.reshape(B, _NP, 1)
    irow = i2.reshape(B, 1, _NP)
    bpad = jnp.pad(pred_boxes, ((0, 0), (0, _QPAD - Q), (0, 0)))
    bT = jnp.transpose(bpad, (0, 2, 1))
    img_h = target_sizes[:, 0].astype(jnp.float32)
    img_w = target_sizes[:, 1].astype(jnp.float32)
    scale = jnp.stack([img_w, img_h], axis=1).reshape(B, 1, 2)

    boxes_pad, smat, labs = pl.pallas_call(
        _iou_body,
        grid=(B,),
        in_specs=[
            pl.BlockSpec((1, _QPAD, 4), lambda b: (b, 0, 0)),
            pl.BlockSpec((1, 4, _QPAD), lambda b: (b, 0, 0)),
            pl.BlockSpec((1, _NP, 1), lambda b: (b, 0, 0)),
            pl.BlockSpec((1, 1, _NP), lambda b: (b, 0, 0)),
            pl.BlockSpec((1, 1, 2), lambda b: (b, 0, 0)),
        ],
        out_specs=[
            pl.BlockSpec((1, _NP, 4), lambda b: (b, 0, 0)),
            pl.BlockSpec((1, _NP, _NP), lambda b: (b, 0, 0)),
            pl.BlockSpec((1, 1, _NP), lambda b: (b, 0, 0)),
        ],
        out_shape=[
            jax.ShapeDtypeStruct((B, _NP, 4), jnp.float32),
            jax.ShapeDtypeStruct((B, _NP, _NP), jnp.float32),
            jax.ShapeDtypeStruct((B, 1, _NP), jnp.int32),
        ],
        compiler_params=pltpu.CompilerParams(
            dimension_semantics=("parallel",)),
    )(bpad, bT, icol, irow, scale)

    keep_pad = pl.pallas_call(
        _nms_scan_body,
        grid=(nsteps,),
        in_specs=[pl.BlockSpec((_IMGS_PER_STEP, _NP, _NP),
                               lambda g: (g, 0, 0))],
        out_specs=pl.BlockSpec((_IMGS_PER_STEP, 1, _NP), lambda g: (g, 0, 0)),
        out_shape=jax.ShapeDtypeStruct((B, 1, _NP), jnp.int32),
        compiler_params=pltpu.CompilerParams(
            dimension_semantics=("parallel",)),
    )(smat)

    scores = vals.reshape(B, _NP)[:, :_NUM_SELECT]
    labels = labs.reshape(B, _NP)[:, :_NUM_SELECT]
    boxes = boxes_pad[:, :_NUM_SELECT, :]
    keep = keep_pad.reshape(B, _NP)[:, :_NUM_SELECT] != 0
    return scores, labels, boxes, keep


# speedup vs baseline: 5.7030x; 2.4583x over previous
import jax
import jax.numpy as jnp
from jax import lax
from jax.experimental import pallas as pl
from jax.experimental.pallas import tpu as pltpu

_NUM_SELECT = 300
_NP = 384
_C = 91
_RW = 320
_RL = 256
_QPAD = 1024
_IOU_THR = 0.5
_IMGS_PER_STEP = 8


def _sigmoid(x):
    return 1.0 / (1.0 + jnp.exp(-x))


def _topk_body(l_ref, lt_ref, vals_ref, idx_ref, *datas):
    for img in range(_IMGS_PER_STEP):
        datas[img][...] = _sigmoid(l_ref[img])
    iota_r = lax.broadcasted_iota(jnp.int32, (1, _RW), 1)
    iota_l = lax.broadcasted_iota(jnp.int32, (1, _RL), 1)

    ms = [_sigmoid(jnp.max(lt_ref[img], axis=0, keepdims=True))
          for img in range(_IMGS_PER_STEP)]

    def body(i, carry):
        ms = list(carry)
        for img in range(_IMGS_PER_STEP):
            m = ms[img]
            data = datas[img]
            vmax = jnp.max(m, axis=1, keepdims=True)
            r = jnp.min(jnp.where(m == vmax, iota_r, _RW))
            row = data[pl.ds(r, 1), :]
            lvec = jnp.min(jnp.where(row == vmax, iota_l, _RL),
                           axis=1, keepdims=True)
            vals_ref[img, pl.ds(i, 1), :] = vmax
            idx_ref[img, pl.ds(i, 1), :] = r * _RL + lvec
            row2 = jnp.where(iota_l == lvec, -1.0, row)
            data[pl.ds(r, 1), :] = row2
            ms[img] = jnp.where(iota_r == r,
                                jnp.max(row2, axis=1, keepdims=True), m)
        return tuple(ms)

    lax.fori_loop(0, _NUM_SELECT, body, tuple(ms))


def _iou_body(b_ref, bt_ref, ic_ref, ir_ref, sc_ref, boxes_ref, s_ref,
              lab_ref):
    qc = ic_ref[0] // _C
    qr = ir_ref[0] // _C
    lab_ref[0] = ir_ref[0] % _C
    bb = b_ref[0]
    bt = bt_ref[0]
    sw = sc_ref[0, 0, 0]
    sh = sc_ref[0, 0, 1]

    onehot_c = (lax.broadcasted_iota(jnp.int32, (_NP, _QPAD), 1) == qc
                ).astype(jnp.float32)
    sel = jnp.dot(onehot_c, bb, preferred_element_type=jnp.float32,
                  precision=lax.Precision.HIGHEST)
    onehot_r = (lax.broadcasted_iota(jnp.int32, (_QPAD, _NP), 0) == qr
                ).astype(jnp.float32)
    selT = jnp.dot(bt, onehot_r, preferred_element_type=jnp.float32,
                   precision=lax.Precision.HIGHEST)

    cx, cy, w, h = sel[:, 0:1], sel[:, 1:2], sel[:, 2:3], sel[:, 3:4]
    x0 = (cx - 0.5 * w) * sw
    y0 = (cy - 0.5 * h) * sh
    x1 = (cx + 0.5 * w) * sw
    y1 = (cy + 0.5 * h) * sh
    cxT, cyT, wT, hT = selT[0:1], selT[1:2], selT[2:3], selT[3:4]
    x0T = (cxT - 0.5 * wT) * sw
    y0T = (cyT - 0.5 * hT) * sh
    x1T = (cxT + 0.5 * wT) * sw
    y1T = (cyT + 0.5 * hT) * sh

    area_c = (x1 - x0) * (y1 - y0)
    area_r = (x1T - x0T) * (y1T - y0T)
    ix0 = jnp.maximum(x0, x0T)
    iy0 = jnp.maximum(y0, y0T)
    ix1 = jnp.minimum(x1, x1T)
    iy1 = jnp.minimum(y1, y1T)
    inter = jnp.clip(ix1 - ix0, 0.0, None) * jnp.clip(iy1 - iy0, 0.0, None)
    union = area_c + area_r - inter
    iou = inter / jnp.clip(union, 1e-9, None)
    s_ref[0] = (iou > _IOU_THR).astype(jnp.float32)
    boxes_ref[0] = jnp.concatenate([x0, y0, x1, y1], axis=1)


def _nms_scan_body(s_ref, keep_ref):
    jiota = lax.broadcasted_iota(jnp.int32, (1, _NP), 1)
    ones = jnp.ones((1, _NP), jnp.float32)

    def body(i, keeps):
        keeps = list(keeps)
        for img in range(_IMGS_PER_STEP):
            keep = keeps[img]
            srow = s_ref[img, pl.ds(i, 1), :]
            supp = jnp.max(jnp.where(jiota < i, keep * srow, 0.0),
                           axis=1, keepdims=True)
            keeps[img] = jnp.where(jiota == i, 1.0 - supp, keep)
        return tuple(keeps)

    keeps = lax.fori_loop(1, _NUM_SELECT, body,
                          tuple(ones for _ in range(_IMGS_PER_STEP)))
    for img in range(_IMGS_PER_STEP):
        keep_ref[img] = keeps[img].astype(jnp.int32)


def kernel(pred_logits, pred_boxes, target_sizes):
    B, Q, C = pred_logits.shape
    nsteps = B // _IMGS_PER_STEP

    flat = pred_logits.reshape(B, Q * C)
    pad = _RW * _RL - Q * C
    flatp = jnp.pad(flat, ((0, 0), (0, pad)), constant_values=-1e30)
    lg = flatp.reshape(B, _RW, _RL)
    lgT = jnp.transpose(lg, (0, 2, 1))

    vals, idx = pl.pallas_call(
        _topk_body,
        grid=(nsteps,),
        in_specs=[
            pl.BlockSpec((_IMGS_PER_STEP, _RW, _RL), lambda g: (g, 0, 0)),
            pl.BlockSpec((_IMGS_PER_STEP, _RL, _RW), lambda g: (g, 0, 0)),
        ],
        out_specs=[pl.BlockSpec((_IMGS_PER_STEP, _NP, 1),
                                lambda g: (g, 0, 0))] * 2,
        out_shape=[
            jax.ShapeDtypeStruct((B, _NP, 1), jnp.float32),
            jax.ShapeDtypeStruct((B, _NP, 1), jnp.int32),
        ],
        scratch_shapes=[pltpu.VMEM((_RW, _RL), jnp.float32)
                        for _ in range(_IMGS_PER_STEP)],
        compiler_params=pltpu.CompilerParams(
            dimension_semantics=("parallel",)),
    )(lg, lgT)

    i2 = idx.reshape(B, _NP)
    icol = i2.reshape(B, _NP, 1)
    irow = i2.reshape(B, 1, _NP)
    bpad = jnp.pad(pred_boxes, ((0, 0), (0, _QPAD - Q), (0, 0)))
    bT = jnp.transpose(bpad, (0, 2, 1))
    img_h = target_sizes[:, 0].astype(jnp.float32)
    img_w = target_sizes[:, 1].astype(jnp.float32)
    scale = jnp.stack([img_w, img_h], axis=1).reshape(B, 1, 2)

    boxes_pad, smat, labs = pl.pallas_call(
        _iou_body,
        grid=(B,),
        in_specs=[
            pl.BlockSpec((1, _QPAD, 4), lambda b: (b, 0, 0)),
            pl.BlockSpec((1, 4, _QPAD), lambda b: (b, 0, 0)),
            pl.BlockSpec((1, _NP, 1), lambda b: (b, 0, 0)),
            pl.BlockSpec((1, 1, _NP), lambda b: (b, 0, 0)),
            pl.BlockSpec((1, 1, 2), lambda b: (b, 0, 0)),
        ],
        out_specs=[
            pl.BlockSpec((1, _NP, 4), lambda b: (b, 0, 0)),
            pl.BlockSpec((1, _NP, _NP), lambda b: (b, 0, 0)),
            pl.BlockSpec((1, 1, _NP), lambda b: (b, 0, 0)),
        ],
        out_shape=[
            jax.ShapeDtypeStruct((B, _NP, 4), jnp.float32),
            jax.ShapeDtypeStruct((B, _NP, _NP), jnp.float32),
            jax.ShapeDtypeStruct((B, 1, _NP), jnp.int32),
        ],
        compiler_params=pltpu.CompilerParams(
            dimension_semantics=("parallel",)),
    )(bpad, bT, icol, irow, scale)

    keep_pad = pl.pallas_call(
        _nms_scan_body,
        grid=(nsteps,),
        in_specs=[pl.BlockSpec((_IMGS_PER_STEP, _NP, _NP),
                               lambda g: (g, 0, 0))],
        out_specs=pl.BlockSpec((_IMGS_PER_STEP, 1, _NP), lambda g: (g, 0, 0)),
        out_shape=jax.ShapeDtypeStruct((B, 1, _NP), jnp.int32),
        compiler_params=pltpu.CompilerParams(
            dimension_semantics=("parallel",)),
    )(smat)

    scores = vals.reshape(B, _NP)[:, :_NUM_SELECT]
    labels = labs.reshape(B, _NP)[:, :_NUM_SELECT]
    boxes = boxes_pad[:, :_NUM_SELECT, :]
    keep = keep_pad.reshape(B, _NP)[:, :_NUM_SELECT] != 0
    return scores, labels, boxes, keep


# speedup vs baseline: 8.8064x; 1.5442x over previous
import jax
import jax.numpy as jnp
from jax import lax
from jax.experimental import pallas as pl
from jax.experimental.pallas import tpu as pltpu

_NUM_SELECT = 300
_NP = 384
_C = 91
_RW = 320
_RL = 256
_QPAD = 1024
_IOU_THR = 0.5
_IMGS_PER_STEP = 16


def _sigmoid(x):
    return 1.0 / (1.0 + jnp.exp(-x))


def _topk_body(l_ref, lt_ref, vals_ref, idx_ref, *datas):
    for img in range(_IMGS_PER_STEP):
        datas[img][...] = _sigmoid(l_ref[img])
    iota_r = lax.broadcasted_iota(jnp.int32, (1, _RW), 1)
    iota_l = lax.broadcasted_iota(jnp.int32, (1, _RL), 1)

    ms = [_sigmoid(jnp.max(lt_ref[img], axis=0, keepdims=True))
          for img in range(_IMGS_PER_STEP)]

    def body(i, carry):
        ms = list(carry)
        for img in range(_IMGS_PER_STEP):
            m = ms[img]
            data = datas[img]
            vmax = jnp.max(m, axis=1, keepdims=True)
            r = jnp.min(jnp.where(m == vmax, iota_r, _RW))
            row = data[pl.ds(r, 1), :]
            lvec = jnp.min(jnp.where(row == vmax, iota_l, _RL),
                           axis=1, keepdims=True)
            vals_ref[img, pl.ds(i, 1), :] = vmax
            idx_ref[img, pl.ds(i, 1), :] = r * _RL + lvec
            row2 = jnp.where(iota_l == lvec, -1.0, row)
            data[pl.ds(r, 1), :] = row2
            ms[img] = jnp.where(iota_r == r,
                                jnp.max(row2, axis=1, keepdims=True), m)
        return tuple(ms)

    lax.fori_loop(0, _NUM_SELECT, body, tuple(ms))


def _iou_body(b_ref, bt_ref, ic_ref, ir_ref, sc_ref, boxes_ref, s_ref,
              lab_ref):
    qc = ic_ref[0] // _C
    qr = ir_ref[0] // _C
    lab_ref[0] = ir_ref[0] % _C
    bb = b_ref[0]
    bt = bt_ref[0]
    sw = sc_ref[0, 0, 0]
    sh = sc_ref[0, 0, 1]

    onehot_c = (lax.broadcasted_iota(jnp.int32, (_NP, _QPAD), 1) == qc
                ).astype(jnp.float32)
    sel = jnp.dot(onehot_c, bb, preferred_element_type=jnp.float32,
                  precision=lax.Precision.HIGHEST)
    onehot_r = (lax.broadcasted_iota(jnp.int32, (_QPAD, _NP), 0) == qr
                ).astype(jnp.float32)
    selT = jnp.dot(bt, onehot_r, preferred_element_type=jnp.float32,
                   precision=lax.Precision.HIGHEST)

    cx, cy, w, h = sel[:, 0:1], sel[:, 1:2], sel[:, 2:3], sel[:, 3:4]
    x0 = (cx - 0.5 * w) * sw
    y0 = (cy - 0.5 * h) * sh
    x1 = (cx + 0.5 * w) * sw
    y1 = (cy + 0.5 * h) * sh
    cxT, cyT, wT, hT = selT[0:1], selT[1:2], selT[2:3], selT[3:4]
    x0T = (cxT - 0.5 * wT) * sw
    y0T = (cyT - 0.5 * hT) * sh
    x1T = (cxT + 0.5 * wT) * sw
    y1T = (cyT + 0.5 * hT) * sh

    area_c = (x1 - x0) * (y1 - y0)
    area_r = (x1T - x0T) * (y1T - y0T)
    ix0 = jnp.maximum(x0, x0T)
    iy0 = jnp.maximum(y0, y0T)
    ix1 = jnp.minimum(x1, x1T)
    iy1 = jnp.minimum(y1, y1T)
    inter = jnp.clip(ix1 - ix0, 0.0, None) * jnp.clip(iy1 - iy0, 0.0, None)
    union = area_c + area_r - inter
    iou = inter / jnp.clip(union, 1e-9, None)
    s_ref[0] = (iou > _IOU_THR).astype(jnp.float32)
    boxes_ref[0] = jnp.concatenate([x0, y0, x1, y1], axis=1)


def _nms_scan_body(s_ref, keep_ref):
    jiota = lax.broadcasted_iota(jnp.int32, (1, _NP), 1)
    ones = jnp.ones((1, _NP), jnp.float32)

    def body(i, keeps):
        keeps = list(keeps)
        for img in range(_IMGS_PER_STEP):
            keep = keeps[img]
            srow = s_ref[img, pl.ds(i, 1), :]
            supp = jnp.max(jnp.where(jiota < i, keep * srow, 0.0),
                           axis=1, keepdims=True)
            keeps[img] = jnp.where(jiota == i, 1.0 - supp, keep)
        return tuple(keeps)

    keeps = lax.fori_loop(1, _NUM_SELECT, body,
                          tuple(ones for _ in range(_IMGS_PER_STEP)))
    for img in range(_IMGS_PER_STEP):
        keep_ref[img] = keeps[img].astype(jnp.int32)


def kernel(pred_logits, pred_boxes, target_sizes):
    B, Q, C = pred_logits.shape
    nsteps = B // _IMGS_PER_STEP

    flat = pred_logits.reshape(B, Q * C)
    pad = _RW * _RL - Q * C
    flatp = jnp.pad(flat, ((0, 0), (0, pad)), constant_values=-1e30)
    lg = flatp.reshape(B, _RW, _RL)
    lgT = jnp.transpose(lg, (0, 2, 1))

    vals, idx = pl.pallas_call(
        _topk_body,
        grid=(nsteps,),
        in_specs=[
            pl.BlockSpec((_IMGS_PER_STEP, _RW, _RL), lambda g: (g, 0, 0)),
            pl.BlockSpec((_IMGS_PER_STEP, _RL, _RW), lambda g: (g, 0, 0)),
        ],
        out_specs=[pl.BlockSpec((_IMGS_PER_STEP, _NP, 1),
                                lambda g: (g, 0, 0))] * 2,
        out_shape=[
            jax.ShapeDtypeStruct((B, _NP, 1), jnp.float32),
            jax.ShapeDtypeStruct((B, _NP, 1), jnp.int32),
        ],
        scratch_shapes=[pltpu.VMEM((_RW, _RL), jnp.float32)
                        for _ in range(_IMGS_PER_STEP)],
        compiler_params=pltpu.CompilerParams(
            dimension_semantics=("parallel",)),
    )(lg, lgT)

    i2 = idx.reshape(B, _NP)
    icol = i2.reshape(B, _NP, 1)
    irow = i2.reshape(B, 1, _NP)
    bpad = jnp.pad(pred_boxes, ((0, 0), (0, _QPAD - Q), (0, 0)))
    bT = jnp.transpose(bpad, (0, 2, 1))
    img_h = target_sizes[:, 0].astype(jnp.float32)
    img_w = target_sizes[:, 1].astype(jnp.float32)
    scale = jnp.stack([img_w, img_h], axis=1).reshape(B, 1, 2)

    boxes_pad, smat, labs = pl.pallas_call(
        _iou_body,
        grid=(B,),
        in_specs=[
            pl.BlockSpec((1, _QPAD, 4), lambda b: (b, 0, 0)),
            pl.BlockSpec((1, 4, _QPAD), lambda b: (b, 0, 0)),
            pl.BlockSpec((1, _NP, 1), lambda b: (b, 0, 0)),
            pl.BlockSpec((1, 1, _NP), lambda b: (b, 0, 0)),
            pl.BlockSpec((1, 1, 2), lambda b: (b, 0, 0)),
        ],
        out_specs=[
            pl.BlockSpec((1, _NP, 4), lambda b: (b, 0, 0)),
            pl.BlockSpec((1, _NP, _NP), lambda b: (b, 0, 0)),
            pl.BlockSpec((1, 1, _NP), lambda b: (b, 0, 0)),
        ],
        out_shape=[
            jax.ShapeDtypeStruct((B, _NP, 4), jnp.float32),
            jax.ShapeDtypeStruct((B, _NP, _NP), jnp.float32),
            jax.ShapeDtypeStruct((B, 1, _NP), jnp.int32),
        ],
        compiler_params=pltpu.CompilerParams(
            dimension_semantics=("parallel",)),
    )(bpad, bT, icol, irow, scale)

    keep_pad = pl.pallas_call(
        _nms_scan_body,
        grid=(nsteps,),
        in_specs=[pl.BlockSpec((_IMGS_PER_STEP, _NP, _NP),
                               lambda g: (g, 0, 0))],
        out_specs=pl.BlockSpec((_IMGS_PER_STEP, 1, _NP), lambda g: (g, 0, 0)),
        out_shape=jax.ShapeDtypeStruct((B, 1, _NP), jnp.int32),
        compiler_params=pltpu.CompilerParams(
            dimension_semantics=("parallel",)),
    )(smat)

    scores = vals.reshape(B, _NP)[:, :_NUM_SELECT]
    labels = labs.reshape(B, _NP)[:, :_NUM_SELECT]
    boxes = boxes_pad[:, :_NUM_SELECT, :]
    keep = keep_pad.reshape(B, _NP)[:, :_NUM_SELECT] != 0
    return scores, labels, boxes, keep
